# Initial kernel scaffold; baseline (speedup 1.0000x reference)
import jax
import jax.numpy as jnp
from jax.experimental import pallas as pl


def kernel(inp, W1, b1, W2, b2, centroids):
    raise NotImplementedError("write your pallas kernel here")


# speedup vs baseline: 1.6772x; 1.6772x over previous
import jax
import jax.numpy as jnp
from jax.experimental import pallas as pl

B, D_IN, H, NZ, NC = 4096, 768, 512, 64, 8192
BLK = 256
GRID = B // BLK


def _body(inp_ref, w1_ref, b1_ref, w2_ref, b2_ref, cent_ref,
          loss_ref, assign_ref, soft_ref, raw_ref):
    i = pl.program_id(0)

    x = inp_ref[...]
    h = jnp.maximum(
        jax.lax.dot_general(x, w1_ref[...], (((1,), (0,)), ((), ())),
                            preferred_element_type=jnp.float32)
        + b1_ref[...], 0.0)
    fv = jax.lax.dot_general(h, w2_ref[...], (((1,), (0,)), ((), ())),
                             preferred_element_type=jnp.float32) \
        + b2_ref[...]

    cent = cent_ref[...]
    dots = jax.lax.dot_general(fv, cent, (((1,), (1,)), ((), ())),
                               preferred_element_type=jnp.float32)
    f_sq = jnp.sum(fv * fv, axis=1, keepdims=True)
    c_sq = jnp.sum(cent * cent, axis=1)[None, :]
    d2 = jnp.maximum(f_sq + c_sq - 2.0 * dots, 0.0)
    d = jnp.sqrt(d2 + 1e-12)

    min_d = jnp.min(d, axis=1)
    cols = jax.lax.broadcasted_iota(jnp.int32, (BLK, NC), 1)
    am = jnp.min(jnp.where(d == min_d[:, None], cols, NC), axis=1)
    assign_ref[...] = am.astype(jnp.int32)

    p = jnp.exp(min_d[:, None] - d)
    z = jnp.sum(p, axis=1)
    soft_add = jnp.sum(p * (1.0 / z)[:, None], axis=0)
    raw_add = jnp.sum((cols == am[:, None]).astype(jnp.int32), axis=0)

    @pl.when(i == 0)
    def _init():
        loss_ref[...] = jnp.zeros_like(loss_ref)
        soft_ref[...] = jnp.zeros_like(soft_ref)
        raw_ref[...] = jnp.zeros_like(raw_ref)

    loss_ref[...] += jnp.sum(min_d).reshape(1, 1)
    soft_ref[...] += soft_add
    raw_ref[...] += raw_add


def kernel(inp, W1, b1, W2, b2, centroids):
    loss_sum, assigns, soft_counts, raw_counts = pl.pallas_call(
        _body,
        grid=(GRID,),
        in_specs=[
            pl.BlockSpec((BLK, D_IN), lambda i: (i, 0)),
            pl.BlockSpec((D_IN, H), lambda i: (0, 0)),
            pl.BlockSpec((1, H), lambda i: (0, 0)),
            pl.BlockSpec((H, NZ), lambda i: (0, 0)),
            pl.BlockSpec((1, NZ), lambda i: (0, 0)),
            pl.BlockSpec((NC, NZ), lambda i: (0, 0)),
        ],
        out_specs=[
            pl.BlockSpec((1, 1), lambda i: (0, 0)),
            pl.BlockSpec((BLK,), lambda i: (i,)),
            pl.BlockSpec((NC,), lambda i: (0,)),
            pl.BlockSpec((NC,), lambda i: (0,)),
        ],
        out_shape=[
            jax.ShapeDtypeStruct((1, 1), jnp.float32),
            jax.ShapeDtypeStruct((B,), jnp.int32),
            jax.ShapeDtypeStruct((NC,), jnp.float32),
            jax.ShapeDtypeStruct((NC,), jnp.int32),
        ],
    )(inp, W1, b1[None, :], W2, b2[None, :], centroids)
    cluster_loss = loss_sum[0, 0] / B
    return (cluster_loss, assigns, soft_counts, raw_counts)


# speedup vs baseline: 1.8383x; 1.0960x over previous
import jax
import jax.numpy as jnp
from jax.experimental import pallas as pl

B, D_IN, H, NZ, NC = 4096, 768, 512, 64, 8192
BLK = 256
GRID = B // BLK


def _body(inp_ref, w1_ref, b1_ref, w2_ref, b2_ref, cent_ref,
          loss_ref, assign_ref, soft_ref, raw_ref):
    i = pl.program_id(0)

    x = inp_ref[...]
    h = jnp.maximum(
        jax.lax.dot_general(x, w1_ref[...], (((1,), (0,)), ((), ())),
                            preferred_element_type=jnp.float32)
        + b1_ref[...], 0.0)
    fv = jax.lax.dot_general(h, w2_ref[...], (((1,), (0,)), ((), ())),
                             preferred_element_type=jnp.float32) \
        + b2_ref[...]

    cent = cent_ref[...]
    dots = jax.lax.dot_general(fv, cent, (((1,), (1,)), ((), ())),
                               preferred_element_type=jnp.float32)
    f_sq = jnp.sum(fv * fv, axis=1, keepdims=True)
    c_sq = jnp.sum(cent * cent, axis=1)[None, :]
    d2 = jnp.maximum(f_sq + c_sq - 2.0 * dots, 0.0)
    d = jnp.sqrt(d2 + 1e-12)

    min_d = jnp.min(d, axis=1)
    cols = jax.lax.broadcasted_iota(jnp.int32, (BLK, NC), 1)
    am = jnp.min(jnp.where(d == min_d[:, None], cols, NC), axis=1)
    assign_ref[...] = am.astype(jnp.int32)

    p = jnp.exp(min_d[:, None] - d)
    z = jnp.sum(p, axis=1)
    soft_add = jax.lax.dot_general((1.0 / z)[None, :], p,
                                   (((1,), (0,)), ((), ())),
                                   preferred_element_type=jnp.float32)[0]
    onehot = (cols == am[:, None]).astype(jnp.float32)
    ones_row = jnp.ones((1, BLK), dtype=jnp.float32)
    raw_add = jax.lax.dot_general(ones_row, onehot,
                                  (((1,), (0,)), ((), ())),
                                  preferred_element_type=jnp.float32)[0]
    raw_add = raw_add.astype(jnp.int32)

    @pl.when(i == 0)
    def _init():
        loss_ref[...] = jnp.zeros_like(loss_ref)
        soft_ref[...] = jnp.zeros_like(soft_ref)
        raw_ref[...] = jnp.zeros_like(raw_ref)

    loss_ref[...] += jnp.sum(min_d).reshape(1, 1)
    soft_ref[...] += soft_add
    raw_ref[...] += raw_add


def kernel(inp, W1, b1, W2, b2, centroids):
    loss_sum, assigns, soft_counts, raw_counts = pl.pallas_call(
        _body,
        grid=(GRID,),
        in_specs=[
            pl.BlockSpec((BLK, D_IN), lambda i: (i, 0)),
            pl.BlockSpec((D_IN, H), lambda i: (0, 0)),
            pl.BlockSpec((1, H), lambda i: (0, 0)),
            pl.BlockSpec((H, NZ), lambda i: (0, 0)),
            pl.BlockSpec((1, NZ), lambda i: (0, 0)),
            pl.BlockSpec((NC, NZ), lambda i: (0, 0)),
        ],
        out_specs=[
            pl.BlockSpec((1, 1), lambda i: (0, 0)),
            pl.BlockSpec((BLK,), lambda i: (i,)),
            pl.BlockSpec((NC,), lambda i: (0,)),
            pl.BlockSpec((NC,), lambda i: (0,)),
        ],
        out_shape=[
            jax.ShapeDtypeStruct((1, 1), jnp.float32),
            jax.ShapeDtypeStruct((B,), jnp.int32),
            jax.ShapeDtypeStruct((NC,), jnp.float32),
            jax.ShapeDtypeStruct((NC,), jnp.int32),
        ],
    )(inp, W1, b1[None, :], W2, b2[None, :], centroids)
    cluster_loss = loss_sum[0, 0] / B
    return (cluster_loss, assigns, soft_counts, raw_counts)
